# baseline (speedup 1.0000x reference)
import functools

import jax
import jax.numpy as jnp
from jax import lax
from jax.experimental import pallas as pl
from jax.experimental.pallas import tpu as pltpu
from jax.experimental.pallas import tpu_sc as plsc

T_ROWS = 8192
DIM = 2048
CHUNK = 8
NBUF = 4


def _sc_gather(idx, table):
    info = plsc.get_sparse_core_info()
    nw = info.num_cores * info.num_subcores
    b_per_w = T_ROWS // nw
    n_chunks = b_per_w // CHUNK

    mesh = plsc.VectorSubcoreMesh(core_axis_name="c", subcore_axis_name="s")

    @functools.partial(
        pl.kernel,
        mesh=mesh,
        out_type=jax.ShapeDtypeStruct((T_ROWS, 1, DIM), jnp.float32),
        scratch_types=[
            pltpu.VMEM((b_per_w,), jnp.int32),
            pltpu.VMEM((NBUF, CHUNK, DIM), jnp.float32),
            pltpu.SemaphoreType.DMA((NBUF,)),
            pltpu.SemaphoreType.DMA((NBUF,)),
        ],
    )
    def body(idx_hbm, table_hbm, out_hbm, idx_v, bufs, gsem, osem):
        wid = lax.axis_index("s") * info.num_cores + lax.axis_index("c")
        base = wid * b_per_w
        pltpu.sync_copy(idx_hbm.at[pl.ds(base, b_per_w)], idx_v)

        def buf_of(c):
            return (c % 2) * 2 + (c // 2) % 2

        def start_gather(c):
            b = buf_of(c)
            return pltpu.async_copy(
                table_hbm.at[idx_v.at[pl.ds(c * CHUNK, CHUNK)]],
                bufs.at[b],
                gsem.at[b],
            )

        def start_out(c):
            b = buf_of(c)
            return pltpu.async_copy(
                bufs.at[b],
                out_hbm.at[pl.ds(base + c * CHUNK, CHUNK), 0],
                osem.at[b],
            )

        g = [None] * n_chunks
        o = [None] * n_chunks
        o_waited = [False] * n_chunks
        for d in range(min(2, n_chunks)):
            g[d] = start_gather(d)
        for c in range(n_chunks):
            g[c].wait()
            o[c] = start_out(c)
            d = c + 2
            if d < n_chunks:
                w = d - 4
                if w >= 0:
                    o[w].wait()
                    o_waited[w] = True
                g[d] = start_gather(d)
        for c in range(n_chunks):
            if not o_waited[c]:
                o[c].wait()

    return body(idx, table)


def kernel(TDist, table):
    idx = TDist.reshape(-1).astype(jnp.int32)
    return _sc_gather(idx, table)


# speedup vs baseline: 1.0220x; 1.0220x over previous
import functools

import jax
import jax.numpy as jnp
from jax import lax
from jax.experimental import pallas as pl
from jax.experimental.pallas import tpu as pltpu
from jax.experimental.pallas import tpu_sc as plsc

T_ROWS = 8192
DIM = 2048
CHUNK = 16
NBUF = 3


def _sc_gather(idx, table):
    info = plsc.get_sparse_core_info()
    nw = info.num_cores * info.num_subcores
    b_per_w = T_ROWS // nw
    n_chunks = b_per_w // CHUNK

    mesh = plsc.VectorSubcoreMesh(core_axis_name="c", subcore_axis_name="s")

    @functools.partial(
        pl.kernel,
        mesh=mesh,
        out_type=jax.ShapeDtypeStruct((T_ROWS, 1, DIM), jnp.float32),
        scratch_types=[
            pltpu.VMEM((b_per_w,), jnp.int32),
            pltpu.VMEM((NBUF, CHUNK, DIM), jnp.float32),
            pltpu.SemaphoreType.DMA((NBUF,)),
            pltpu.SemaphoreType.DMA((NBUF,)),
        ],
    )
    def body(idx_hbm, table_hbm, out_hbm, idx_v, bufs, gsem, osem):
        wid = lax.axis_index("s") * info.num_cores + lax.axis_index("c")
        base = wid * b_per_w
        pltpu.sync_copy(idx_hbm.at[pl.ds(base, b_per_w)], idx_v)

        def start_gather(c):
            b = c % NBUF
            return pltpu.async_copy(
                table_hbm.at[idx_v.at[pl.ds(c * CHUNK, CHUNK)]],
                bufs.at[b],
                gsem.at[b],
            )

        def start_out(c):
            b = c % NBUF
            return pltpu.async_copy(
                bufs.at[b],
                out_hbm.at[pl.ds(base + c * CHUNK, CHUNK), 0],
                osem.at[b],
            )

        depth = NBUF - 1
        g = [None] * n_chunks
        o = [None] * n_chunks
        o_waited = [False] * n_chunks
        for d in range(min(depth, n_chunks)):
            g[d] = start_gather(d)
        for c in range(n_chunks):
            g[c].wait()
            o[c] = start_out(c)
            d = c + depth
            if d < n_chunks:
                w = d - NBUF
                if w >= 0:
                    o[w].wait()
                    o_waited[w] = True
                g[d] = start_gather(d)
        for c in range(n_chunks):
            if not o_waited[c]:
                o[c].wait()

    return body(idx, table)


def kernel(TDist, table):
    idx = TDist.reshape(-1).astype(jnp.int32)
    return _sc_gather(idx, table)
